# Initial kernel scaffold; baseline (speedup 1.0000x reference)
import jax
import jax.numpy as jnp
from jax.experimental import pallas as pl


def kernel(x_user, x_item, ei_u2i, ei_i2u, et_u2i, et_i2u, W0_u2i, b0_u2i, W0_i2u, b0_i2u, W1_u2i, b1_u2i, W1_i2u, b1_i2u, We_u2i, be_u2i, We_i2u, be_i2u, g_user, bt_user, g_item, bt_item):
    raise NotImplementedError("write your pallas kernel here")


# speedup vs baseline: 3.1497x; 3.1497x over previous
import functools

import jax
import jax.numpy as jnp
from jax import lax
from jax.experimental import pallas as pl
from jax.experimental.pallas import tpu as pltpu
from jax.experimental.pallas import tpu_sc as plsc

N = 50000
E = 300000
D = 128
ED = 16

NS = 16
CHUNK = 512
EPAD = 303104
IDXROWS = EPAD // 128
ROWS_PER_TILE = IDXROWS // NS
CHUNKS = ROWS_PER_TILE // 4
NPAD = 50048
STRIPE = NPAD // NS
DUMMY = NPAD - 1

_mesh = plsc.VectorSubcoreMesh(core_axis_name="c", subcore_axis_name="s")


def _zero_stripe(zbuf, acc, base):
    for k in range(24):
        pltpu.sync_copy(zbuf, acc.at[pl.ds(base + k * 128, 128)])
    pltpu.sync_copy(zbuf.at[pl.ds(0, 56)], acc.at[pl.ds(base + 24 * 128, 56)])


@functools.partial(
    pl.kernel,
    out_type=[jax.ShapeDtypeStruct((NPAD, ED), jnp.float32)] * 4,
    mesh=_mesh,
    compiler_params=pltpu.CompilerParams(use_tc_tiling_on_sc=False),
    scratch_types=[
        pltpu.VMEM((4, 128), jnp.int32),
        pltpu.VMEM((CHUNK, ED), jnp.float32),
        pltpu.VMEM((128, ED), jnp.float32),
        pltpu.VMEM((128, ED), jnp.float32),
        pltpu.VMEM_SHARED((NPAD, ED), jnp.float32),
        pltpu.VMEM_SHARED((NPAD, ED), jnp.float32),
    ],
)
def _sc_precompute(dst_a, et_a, dst_b, et_b, ones_hbm, z16_hbm,
                   ets_a_out, cnt_a_out, ets_b_out, cnt_b_out,
                   dstidx_v, etbuf_v, ones_v, zbuf_v, ets_acc, cnt_acc):
    core = lax.axis_index("c")
    sub = lax.axis_index("s")
    pltpu.sync_copy(ones_hbm, ones_v)
    pltpu.sync_copy(z16_hbm, zbuf_v)
    base = sub * STRIPE

    def run(dst_r, et_r, ets_out, cnt_out):
        _zero_stripe(zbuf_v, ets_acc, base)
        _zero_stripe(zbuf_v, cnt_acc, base)
        plsc.subcore_barrier()

        def chunk_body(c, carry):
            r = sub * ROWS_PER_TILE + c * 4
            pltpu.sync_copy(dst_r.at[pl.ds(r, 4)], dstidx_v)
            pltpu.sync_copy(et_r.at[pl.ds(r * 128, CHUNK)], etbuf_v)
            for j in range(4):
                pltpu.sync_copy(etbuf_v.at[pl.ds(j * 128, 128)],
                                ets_acc.at[dstidx_v.at[j]], add=True)
                pltpu.sync_copy(ones_v, cnt_acc.at[dstidx_v.at[j]], add=True)
            return carry

        lax.fori_loop(0, CHUNKS, chunk_body, 0)
        plsc.subcore_barrier()
        pltpu.sync_copy(ets_acc.at[pl.ds(base, STRIPE)],
                        ets_out.at[pl.ds(base, STRIPE)])
        pltpu.sync_copy(cnt_acc.at[pl.ds(base, STRIPE)],
                        cnt_out.at[pl.ds(base, STRIPE)])

    pl.when(core == 0)(lambda: run(dst_a, et_a, ets_a_out, cnt_a_out))
    pl.when(core == 1)(lambda: run(dst_b, et_b, ets_b_out, cnt_b_out))


@functools.partial(
    pl.kernel,
    out_type=[jax.ShapeDtypeStruct((4, NPAD, 32), jnp.float32)] * 2,
    mesh=_mesh,
    compiler_params=pltpu.CompilerParams(use_tc_tiling_on_sc=False),
    scratch_types=[
        pltpu.VMEM((4, 128), jnp.int32),
        pltpu.VMEM((4, 128), jnp.int32),
        pltpu.VMEM((CHUNK, 32), jnp.float32),
        pltpu.VMEM((128, 32), jnp.float32),
        pltpu.VMEM_SHARED((NPAD, 32), jnp.float32),
        pltpu.SemaphoreType.DMA,
    ],
)
def _sc_agg(hu4, hi4, src_u, dst_u, src_i, dst_i, z32_hbm,
            agg_u_out, agg_i_out,
            srcidx_v, dstidx_v, rows_v, zbuf_v, acc, sem):
    core = lax.axis_index("c")
    sub = lax.axis_index("s")
    pltpu.sync_copy(z32_hbm, zbuf_v)
    base = sub * STRIPE

    def run(h4, src4_r, dst_r, out):
        for p in range(4):
            _zero_stripe(zbuf_v, acc, base)
            plsc.subcore_barrier()

            def chunk_body(c, carry):
                r = sub * ROWS_PER_TILE + c * 4
                pltpu.sync_copy(src4_r.at[p, pl.ds(r, 4)], srcidx_v)
                pltpu.sync_copy(dst_r.at[pl.ds(r, 4)], dstidx_v)
                cps = [
                    pltpu.async_copy(h4.at[srcidx_v.at[j]],
                                     rows_v.at[pl.ds(j * 128, 128)], sem)
                    for j in range(4)
                ]
                for j in range(4):
                    cps[j].wait()
                for j in range(4):
                    pltpu.sync_copy(rows_v.at[pl.ds(j * 128, 128)],
                                    acc.at[dstidx_v.at[j]], add=True)
                return carry

            lax.fori_loop(0, CHUNKS, chunk_body, 0)
            plsc.subcore_barrier()
            pltpu.sync_copy(acc.at[pl.ds(base, STRIPE)],
                            out.at[p, pl.ds(base, STRIPE)])

    pl.when(core == 0)(lambda: run(hu4, src_u, dst_u, agg_u_out))
    pl.when(core == 1)(lambda: run(hi4, src_i, dst_i, agg_i_out))


BM = 400


def _combine_body(hd, agg, cnt, ets, wt, wb, bv, we, bev, g, bt, out):
    c = cnt[:, 0:1]
    recip = 1.0 / jnp.maximum(c, 1.0)
    a = jnp.concatenate([agg[0], agg[1], agg[2], agg[3]], axis=-1) * recip
    y = (jnp.dot(hd[...], wt[...], preferred_element_type=jnp.float32)
         + jnp.dot(a, wb[...], preferred_element_type=jnp.float32)
         + bv[...])
    add = (jnp.dot(ets[...] * recip, we[...],
                   preferred_element_type=jnp.float32)
           + jnp.where(c > 0.0, 1.0, 0.0) * bev[...])
    t = y + add
    m = jnp.mean(t, axis=-1, keepdims=True)
    v = jnp.mean((t - m) ** 2, axis=-1, keepdims=True)
    out[...] = (t - m) * lax.rsqrt(v + 1e-5) * g[...] + bt[...]


def _combine(hd, agg, cnt, ets, w, b, we, be, g, bt):
    full = lambda i: (0, 0)
    return pl.pallas_call(
        _combine_body,
        grid=(N // BM,),
        in_specs=[
            pl.BlockSpec((BM, D), lambda i: (i, 0)),
            pl.BlockSpec((4, BM, 32), lambda i: (0, i, 0)),
            pl.BlockSpec((BM, ED), lambda i: (i, 0)),
            pl.BlockSpec((BM, ED), lambda i: (i, 0)),
            pl.BlockSpec((D, D), full),
            pl.BlockSpec((D, D), full),
            pl.BlockSpec((1, D), full),
            pl.BlockSpec((ED, D), full),
            pl.BlockSpec((1, D), full),
            pl.BlockSpec((1, D), full),
            pl.BlockSpec((1, D), full),
        ],
        out_specs=pl.BlockSpec((BM, D), lambda i: (i, 0)),
        out_shape=jax.ShapeDtypeStruct((N, D), jnp.float32),
    )(hd, agg, cnt, ets, w[:D], w[D:], b.reshape(1, D), we,
      be.reshape(1, D), g.reshape(1, D), bt.reshape(1, D))


def _pad_idx(src, dst):
    srcp = jnp.concatenate([src, jnp.zeros((EPAD - E,), jnp.int32)])
    dstp = jnp.concatenate(
        [dst, jnp.full((EPAD - E,), DUMMY, jnp.int32)])
    src4 = (srcp[None, :] * 4
            + jnp.arange(4, dtype=jnp.int32)[:, None]).reshape(4, IDXROWS, 128)
    return src4, dstp.reshape(IDXROWS, 128)


@jax.jit
def kernel(x_user, x_item, ei_u2i, ei_i2u, et_u2i, et_i2u,
           W0_u2i, b0_u2i, W0_i2u, b0_i2u,
           W1_u2i, b1_u2i, W1_i2u, b1_i2u,
           We_u2i, be_u2i, We_i2u, be_i2u,
           g_user, bt_user, g_item, bt_item):
    src4_u, dstp_u = _pad_idx(ei_u2i[0], ei_u2i[1])
    src4_i, dstp_i = _pad_idx(ei_i2u[0], ei_i2u[1])
    zpad = jnp.zeros((EPAD - E, ED), jnp.float32)
    et_u_p = jnp.concatenate([et_u2i, zpad])
    et_i_p = jnp.concatenate([et_i2u, zpad])
    ones16 = jnp.zeros((128, ED), jnp.float32).at[:, 0].set(1.0)
    z16 = jnp.zeros((128, ED), jnp.float32)
    z32 = jnp.zeros((128, 32), jnp.float32)

    ets_u, cnt_u, ets_i, cnt_i = _sc_precompute(
        dstp_u, et_u_p, dstp_i, et_i_p, ones16, z16)

    h_u, h_i = x_user, x_item
    for (wu, bu, wi, bi) in ((W0_u2i, b0_u2i, W0_i2u, b0_i2u),
                             (W1_u2i, b1_u2i, W1_i2u, b1_i2u)):
        agg_u2i, agg_i2u = _sc_agg(
            h_u.reshape(4 * N, 32), h_i.reshape(4 * N, 32),
            src4_u, dstp_u, src4_i, dstp_i, z32)
        h_i_new = _combine(h_i, agg_u2i, cnt_u, ets_u, wu, bu,
                           We_u2i, be_u2i, g_item, bt_item)
        h_u_new = _combine(h_u, agg_i2u, cnt_i, ets_i, wi, bi,
                           We_i2u, be_i2u, g_user, bt_user)
        h_u, h_i = h_u_new, h_i_new
    return h_u, h_i
